# Initial kernel scaffold; baseline (speedup 1.0000x reference)
import jax
import jax.numpy as jnp
from jax import lax
from jax.experimental import pallas as pl
from jax.experimental.pallas import tpu as pltpu
from jax.experimental.pallas import tpu_sc as plsc

NC = 2
NS = 16
CH = 80


def _mlp_body(node_ref, Ws_ref, bs_ref, Wphi_ref, bphi_ref,
              s1_0, s1_1, s23_0, s23_1):
    x = node_ref[...]
    h = lax.dot_general(x, Ws_ref[...], (((1,), (1,)), ((), ())),
                        preferred_element_type=jnp.float32)
    h = h + bs_ref[...][None, :]
    h = h * jax.nn.sigmoid(h)
    s = lax.dot_general(h, Wphi_ref[...], (((1,), (1,)), ((), ())),
                        preferred_element_type=jnp.float32)
    s = s + bphi_ref[...][None, :]
    s1_0[...] = s[:, 0:64]
    s1_1[...] = s[:, 64:128]
    s23_0[...] = jnp.concatenate([s[:, 128:192], s[:, 256:320]], axis=1)
    s23_1[...] = jnp.concatenate([s[:, 192:256], s[:, 320:384]], axis=1)


def _w_body(rbf_ref, env_ref, Ww_ref, bw_ref, w1_0, w1_1, w23_0, w23_1):
    w = lax.dot_general(rbf_ref[...], Ww_ref[...], (((1,), (1,)), ((), ())),
                        preferred_element_type=jnp.float32)
    w = (w + bw_ref[...][None, :]) * env_ref[...][:, None]
    w1_0[...] = w[:, 0:64]
    w1_1[...] = w[:, 64:128]
    w23_0[...] = jnp.concatenate([w[:, 128:192], w[:, 256:320]], axis=1)
    w23_1[...] = jnp.concatenate([w[:, 192:256], w[:, 320:384]], axis=1)


def _dv_body(jj, ii, rij, zeros, eq_0, eq_1, s23_0, s23_1, w23_0, w23_1,
             out_0, out_1,
             jv, iv, eqbuf, s23buf, w23buf, rijbuf, paybuf, acc, sem1, sem2):
    c = lax.axis_index("c")
    s = lax.axis_index("s")
    rows = zeros.shape[0]
    e_total = jj.shape[0]
    epw = e_total // NS
    nchunk = epw // CH

    def run(eq_t, s23_t, w23_t, out_t):
        pltpu.sync_copy(zeros, acc.at[pl.ds(s * rows, rows)])
        plsc.subcore_barrier()

        def chunk(k, carry):
            off = s * epw + k * CH
            pltpu.sync_copy(jj.at[pl.ds(off, CH)], jv)
            pltpu.sync_copy(ii.at[pl.ds(off, CH)], iv)
            cp1 = pltpu.async_copy(eq_t.at[jv], eqbuf, sem1)
            cp2 = pltpu.async_copy(s23_t.at[jv], s23buf, sem2)
            pltpu.sync_copy(w23_t.at[pl.ds(off, CH)], w23buf)
            pltpu.sync_copy(rij.at[pl.ds(off * 3, CH * 3)], rijbuf)
            cp1.wait()
            cp2.wait()

            def edge(e, ecarry):
                rd = [plsc.load_gather(rijbuf,
                                       [jnp.full((16,), e * 3 + d, jnp.int32)])
                      for d in range(3)]
                for q in range(4):
                    sw2 = (s23buf[e, pl.ds(q * 16, 16)]
                           * w23buf[e, pl.ds(q * 16, 16)])
                    sw3 = (s23buf[e, pl.ds(64 + q * 16, 16)]
                           * w23buf[e, pl.ds(64 + q * 16, 16)])
                    for d in range(3):
                        o = d * 64 + q * 16
                        paybuf[e, pl.ds(o, 16)] = (
                            eqbuf[e, pl.ds(o, 16)] * sw2 + rd[d] * sw3)
                return ecarry

            lax.fori_loop(0, CH, edge, 0)
            pltpu.sync_copy(paybuf, acc.at[iv], add=True)
            return carry

        lax.fori_loop(0, nchunk, chunk, 0)
        plsc.subcore_barrier()
        pltpu.sync_copy(acc.at[pl.ds(s * rows, rows)],
                        out_t.at[pl.ds(s * rows, rows)])

    @pl.when(c == 0)
    def _():
        run(eq_0, s23_0, w23_0, out_0)

    @pl.when(c == 1)
    def _():
        run(eq_1, s23_1, w23_1, out_1)


def _ds_body(jj, ii, zeros, s1_0, s1_1, w1_0, w1_1,
             out_0, out_1,
             jv, iv, s1buf, w1buf, paybuf, acc, sem1):
    c = lax.axis_index("c")
    s = lax.axis_index("s")
    rows = zeros.shape[0]
    e_total = jj.shape[0]
    epw = e_total // NS
    nchunk = epw // CH

    def run(s1_t, w1_t, out_t):
        pltpu.sync_copy(zeros, acc.at[pl.ds(s * rows, rows)])
        plsc.subcore_barrier()

        def chunk(k, carry):
            off = s * epw + k * CH
            pltpu.sync_copy(jj.at[pl.ds(off, CH)], jv)
            pltpu.sync_copy(ii.at[pl.ds(off, CH)], iv)
            cp1 = pltpu.async_copy(s1_t.at[jv], s1buf, sem1)
            pltpu.sync_copy(w1_t.at[pl.ds(off, CH)], w1buf)
            cp1.wait()

            def edge(e, ecarry):
                for q in range(4):
                    paybuf[e, pl.ds(q * 16, 16)] = (
                        s1buf[e, pl.ds(q * 16, 16)]
                        * w1buf[e, pl.ds(q * 16, 16)])
                return ecarry

            lax.fori_loop(0, CH, edge, 0)
            pltpu.sync_copy(paybuf, acc.at[iv], add=True)
            return carry

        lax.fori_loop(0, nchunk, chunk, 0)
        plsc.subcore_barrier()
        pltpu.sync_copy(acc.at[pl.ds(s * rows, rows)],
                        out_t.at[pl.ds(s * rows, rows)])

    @pl.when(c == 0)
    def _():
        run(s1_0, w1_0, out_0)

    @pl.when(c == 1)
    def _():
        run(s1_1, w1_1, out_1)


def kernel(node, equivariant, rbf, envelope, r_ij, edge_index,
           Ws, bs, Wphi, bphi, Ww, bw):
    n, units = node.shape
    e = rbf.shape[0]
    f = jnp.float32

    nb = 2000
    s_tabs = pl.pallas_call(
        _mlp_body,
        grid=(n // nb,),
        in_specs=[
            pl.BlockSpec((nb, units), lambda i: (i, 0)),
            pl.BlockSpec((units, units), lambda i: (0, 0)),
            pl.BlockSpec((units,), lambda i: (0,)),
            pl.BlockSpec((3 * units, units), lambda i: (0, 0)),
            pl.BlockSpec((3 * units,), lambda i: (0,)),
        ],
        out_specs=[
            pl.BlockSpec((nb, 64), lambda i: (i, 0)),
            pl.BlockSpec((nb, 64), lambda i: (i, 0)),
            pl.BlockSpec((nb, 128), lambda i: (i, 0)),
            pl.BlockSpec((nb, 128), lambda i: (i, 0)),
        ],
        out_shape=[
            jax.ShapeDtypeStruct((n, 64), f),
            jax.ShapeDtypeStruct((n, 64), f),
            jax.ShapeDtypeStruct((n, 128), f),
            jax.ShapeDtypeStruct((n, 128), f),
        ],
    )(node, Ws, bs, Wphi, bphi)
    s1_0, s1_1, s23_0, s23_1 = s_tabs

    eb = 4000
    nr = rbf.shape[1]
    env1 = envelope.reshape(e)
    w_tabs = pl.pallas_call(
        _w_body,
        grid=(e // eb,),
        in_specs=[
            pl.BlockSpec((eb, nr), lambda i: (i, 0)),
            pl.BlockSpec((eb,), lambda i: (i,)),
            pl.BlockSpec((3 * units, nr), lambda i: (0, 0)),
            pl.BlockSpec((3 * units,), lambda i: (0,)),
        ],
        out_specs=[
            pl.BlockSpec((eb, 64), lambda i: (i, 0)),
            pl.BlockSpec((eb, 64), lambda i: (i, 0)),
            pl.BlockSpec((eb, 128), lambda i: (i, 0)),
            pl.BlockSpec((eb, 128), lambda i: (i, 0)),
        ],
        out_shape=[
            jax.ShapeDtypeStruct((e, 64), f),
            jax.ShapeDtypeStruct((e, 64), f),
            jax.ShapeDtypeStruct((e, 128), f),
            jax.ShapeDtypeStruct((e, 128), f),
        ],
    )(rbf, env1, Ww, bw)
    w1_0, w1_1, w23_0, w23_1 = w_tabs

    ii = edge_index[:, 0]
    jj = edge_index[:, 1]
    eq_0 = equivariant[:, :, 0:64].reshape(n, 192)
    eq_1 = equivariant[:, :, 64:128].reshape(n, 192)
    rij_flat = r_ij.reshape(e * 3)
    zeros192 = jnp.zeros((n // NS, 192), f)
    zeros64 = jnp.zeros((n // NS, 64), f)

    mesh = plsc.VectorSubcoreMesh(core_axis_name="c", subcore_axis_name="s")

    dv_call = pl.kernel(
        _dv_body,
        out_type=(jax.ShapeDtypeStruct((n, 192), f),
                  jax.ShapeDtypeStruct((n, 192), f)),
        mesh=mesh,
        scratch_types=[
            pltpu.VMEM((CH,), jnp.int32),
            pltpu.VMEM((CH,), jnp.int32),
            pltpu.VMEM((CH, 192), f),
            pltpu.VMEM((CH, 128), f),
            pltpu.VMEM((CH, 128), f),
            pltpu.VMEM((CH * 3,), f),
            pltpu.VMEM((CH, 192), f),
            pltpu.VMEM_SHARED((n, 192), f),
            pltpu.SemaphoreType.DMA,
            pltpu.SemaphoreType.DMA,
        ],
    )
    dv_0, dv_1 = dv_call(jj, ii, rij_flat, zeros192,
                         eq_0, eq_1, s23_0, s23_1, w23_0, w23_1)

    ds_call = pl.kernel(
        _ds_body,
        out_type=(jax.ShapeDtypeStruct((n, 64), f),
                  jax.ShapeDtypeStruct((n, 64), f)),
        mesh=mesh,
        scratch_types=[
            pltpu.VMEM((CH,), jnp.int32),
            pltpu.VMEM((CH,), jnp.int32),
            pltpu.VMEM((CH, 64), f),
            pltpu.VMEM((CH, 64), f),
            pltpu.VMEM((CH, 64), f),
            pltpu.VMEM_SHARED((n, 64), f),
            pltpu.SemaphoreType.DMA,
        ],
    )
    ds_0, ds_1 = ds_call(jj, ii, zeros64, s1_0, s1_1, w1_0, w1_1)

    ds = jnp.concatenate([ds_0, ds_1], axis=1)
    dv = jnp.concatenate([dv_0.reshape(n, 3, 64), dv_1.reshape(n, 3, 64)],
                         axis=2)
    return (ds, dv)


# speedup vs baseline: 6.6313x; 6.6313x over previous
import jax
import jax.numpy as jnp
from jax import lax
from jax.experimental import pallas as pl
from jax.experimental.pallas import tpu as pltpu
from jax.experimental.pallas import tpu_sc as plsc

NC = 2
NS = 16
CH = 40


def _mlp_body(node_ref, eqf_ref, Ws_ref, bs_ref, Wphi_ref, bphi_ref, *outs):
    x = node_ref[...]
    h = lax.dot_general(x, Ws_ref[...], (((1,), (1,)), ((), ())),
                        preferred_element_type=jnp.float32)
    h = h + bs_ref[...][None, :]
    h = h * jax.nn.sigmoid(h)
    s = lax.dot_general(h, Wphi_ref[...], (((1,), (1,)), ((), ())),
                        preferred_element_type=jnp.float32)
    s = s + bphi_ref[...][None, :]
    eqf = eqf_ref[...]
    pad = jnp.zeros((x.shape[0], 64), jnp.float32)
    for q in range(4):
        o = 32 * q
        outs[q][...] = jnp.concatenate(
            [eqf[:, 0, o:o + 32], eqf[:, 1, o:o + 32], eqf[:, 2, o:o + 32],
             s[:, o:o + 32], s[:, 128 + o:128 + o + 32],
             s[:, 256 + o:256 + o + 32], pad], axis=1)


def _w_body(rbf_ref, env_ref, rij_ref, Ww_ref, bw_ref, w0, w1, w2, w3, r_out):
    w = lax.dot_general(rbf_ref[...], Ww_ref[...], (((1,), (1,)), ((), ())),
                        preferred_element_type=jnp.float32)
    w = (w + bw_ref[...][None, :]) * env_ref[...]
    ws = (w0, w1, w2, w3)
    for q in range(4):
        o = 32 * q
        ws[q][...] = jnp.concatenate(
            [w[:, o:o + 32], w[:, 128 + o:128 + o + 32],
             w[:, 256 + o:256 + o + 32]], axis=1)
    r = rij_ref[...]
    eb = r.shape[0]
    r_out[...] = jnp.concatenate(
        [jnp.broadcast_to(r[:, d:d + 1], (eb, 16)) for d in range(3)], axis=1)


def _sc_body(jj, ii, rr, zeros, t0, t1, t2, t3, w0, w1, w2, w3,
             out0, out1, out2, out3,
             jv, iv, tbuf, wbuf, rbuf, pay, acc, sem1):
    c = lax.axis_index("c")
    s = lax.axis_index("s")
    rows = zeros.shape[0]
    epw = jj.shape[0] // NS
    nchunk = epw // CH

    def task(t_t, w_t, out_t):
        plsc.subcore_barrier()
        pltpu.sync_copy(zeros, acc.at[pl.ds(s * rows, rows)])
        plsc.subcore_barrier()

        def chunk(k, carry):
            off = s * epw + k * CH
            pltpu.sync_copy(jj.at[pl.ds(off, CH)], jv)
            pltpu.sync_copy(ii.at[pl.ds(off, CH)], iv)
            gather = pltpu.async_copy(t_t.at[jv], tbuf, sem1)
            pltpu.sync_copy(w_t.at[pl.ds(off, CH)], wbuf)
            pltpu.sync_copy(rr.at[pl.ds(off, CH)], rbuf)
            gather.wait()

            def edge(e, ecarry):
                rd = [rbuf[e, pl.ds(16 * d, 16)] for d in range(3)]
                for u in range(2):
                    o = 16 * u
                    s1v = tbuf[e, pl.ds(96 + o, 16)]
                    sw2 = (tbuf[e, pl.ds(128 + o, 16)]
                           * wbuf[e, pl.ds(32 + o, 16)])
                    sw3 = (tbuf[e, pl.ds(160 + o, 16)]
                           * wbuf[e, pl.ds(64 + o, 16)])
                    pay[e, pl.ds(o, 16)] = s1v * wbuf[e, pl.ds(o, 16)]
                    for d in range(3):
                        pay[e, pl.ds(32 + 32 * d + o, 16)] = (
                            tbuf[e, pl.ds(32 * d + o, 16)] * sw2
                            + rd[d] * sw3)
                return ecarry

            lax.fori_loop(0, CH, edge, 0)
            pltpu.sync_copy(pay, acc.at[iv], add=True)
            return carry

        lax.fori_loop(0, nchunk, chunk, 0)
        plsc.subcore_barrier()
        pltpu.sync_copy(acc.at[pl.ds(s * rows, rows)],
                        out_t.at[pl.ds(s * rows, rows)])

    @pl.when(c == 0)
    def _():
        task(t0, w0, out0)
        task(t2, w2, out2)

    @pl.when(c == 1)
    def _():
        task(t1, w1, out1)
        task(t3, w3, out3)


def kernel(node, equivariant, rbf, envelope, r_ij, edge_index,
           Ws, bs, Wphi, bphi, Ww, bw):
    n, units = node.shape
    e = rbf.shape[0]
    f = jnp.float32

    nb = 2000
    t_tabs = pl.pallas_call(
        _mlp_body,
        grid=(n // nb,),
        in_specs=[
            pl.BlockSpec((nb, units), lambda i: (i, 0)),
            pl.BlockSpec((nb, 3, units), lambda i: (i, 0, 0)),
            pl.BlockSpec((units, units), lambda i: (0, 0)),
            pl.BlockSpec((units,), lambda i: (0,)),
            pl.BlockSpec((3 * units, units), lambda i: (0, 0)),
            pl.BlockSpec((3 * units,), lambda i: (0,)),
        ],
        out_specs=[pl.BlockSpec((nb, 256), lambda i: (i, 0))] * 4,
        out_shape=[jax.ShapeDtypeStruct((n, 256), f)] * 4,
    )(node, equivariant, Ws, bs, Wphi, bphi)

    eb = 4000
    nr = rbf.shape[1]
    w_tabs = pl.pallas_call(
        _w_body,
        grid=(e // eb,),
        in_specs=[
            pl.BlockSpec((eb, nr), lambda i: (i, 0)),
            pl.BlockSpec((eb, 1), lambda i: (i, 0)),
            pl.BlockSpec((eb, 3), lambda i: (i, 0)),
            pl.BlockSpec((3 * units, nr), lambda i: (0, 0)),
            pl.BlockSpec((3 * units,), lambda i: (0,)),
        ],
        out_specs=[pl.BlockSpec((eb, 96), lambda i: (i, 0))] * 4
        + [pl.BlockSpec((eb, 48), lambda i: (i, 0))],
        out_shape=[jax.ShapeDtypeStruct((e, 96), f)] * 4
        + [jax.ShapeDtypeStruct((e, 48), f)],
    )(rbf, envelope, r_ij, Ww, bw)
    w0, w1, w2, w3, rr = w_tabs

    ii = edge_index[:, 0]
    jj = edge_index[:, 1]
    npad = ((n + NS * 8 - 1) // (NS * 8)) * NS * 8
    zeros = jnp.zeros((npad // NS, 128), f)

    mesh = plsc.VectorSubcoreMesh(core_axis_name="c", subcore_axis_name="s")
    sc_call = pl.kernel(
        _sc_body,
        out_type=tuple(jax.ShapeDtypeStruct((npad, 128), f) for _ in range(4)),
        mesh=mesh,
        scratch_types=[
            pltpu.VMEM((CH,), jnp.int32),
            pltpu.VMEM((CH,), jnp.int32),
            pltpu.VMEM((CH, 256), f),
            pltpu.VMEM((CH, 96), f),
            pltpu.VMEM((CH, 48), f),
            pltpu.VMEM((CH, 128), f),
            pltpu.VMEM_SHARED((npad, 128), f),
            pltpu.SemaphoreType.DMA,
        ],
    )
    outs = sc_call(jj, ii, rr, zeros, *t_tabs, w0, w1, w2, w3)

    ds = jnp.concatenate([outs[q][:n, 0:32] for q in range(4)], axis=1)
    dv = jnp.stack(
        [jnp.concatenate([outs[q][:n, 32 + 32 * d:64 + 32 * d]
                          for q in range(4)], axis=1) for d in range(3)],
        axis=1)
    return (ds, dv)


# speedup vs baseline: 16.9349x; 2.5538x over previous
import jax
import jax.numpy as jnp
from jax import lax
from jax.experimental import pallas as pl
from jax.experimental.pallas import tpu as pltpu
from jax.experimental.pallas import tpu_sc as plsc

NC = 2
NS = 16
CH = 40


def _mlp_body(node_ref, eqf_ref, Ws_ref, bs_ref, Wphi_ref, bphi_ref, *outs):
    x = node_ref[...]
    h = lax.dot_general(x, Ws_ref[...], (((1,), (1,)), ((), ())),
                        preferred_element_type=jnp.float32)
    h = h + bs_ref[...][None, :]
    h = h * jax.nn.sigmoid(h)
    s = lax.dot_general(h, Wphi_ref[...], (((1,), (1,)), ((), ())),
                        preferred_element_type=jnp.float32)
    s = s + bphi_ref[...][None, :]
    eqf = eqf_ref[...]
    pad = jnp.zeros((x.shape[0], 64), jnp.float32)
    for q in range(4):
        o = 32 * q
        outs[q][...] = jnp.concatenate(
            [eqf[:, 0, o:o + 32], eqf[:, 1, o:o + 32], eqf[:, 2, o:o + 32],
             s[:, o:o + 32], s[:, 128 + o:128 + o + 32],
             s[:, 256 + o:256 + o + 32], pad], axis=1)


def _w_body(rbf_ref, env_ref, rij_ref, Ww_ref, bw_ref, w0, w1, w2, w3, r_out):
    w = lax.dot_general(rbf_ref[...], Ww_ref[...], (((1,), (1,)), ((), ())),
                        preferred_element_type=jnp.float32)
    w = (w + bw_ref[...][None, :]) * env_ref[...]
    ws = (w0, w1, w2, w3)
    for q in range(4):
        o = 32 * q
        ws[q][...] = jnp.concatenate(
            [w[:, o:o + 32], w[:, 128 + o:128 + o + 32],
             w[:, 256 + o:256 + o + 32]], axis=1)
    r = rij_ref[...]
    eb = r.shape[0]
    r_out[...] = jnp.concatenate(
        [jnp.broadcast_to(r[:, d:d + 1], (eb, 16)) for d in range(3)], axis=1)


def _sc_body(jj, ii, rr, zeros, t0, t1, t2, t3, w0, w1, w2, w3,
             out0, out1, out2, out3,
             jvb0, jvb1, ivb0, ivb1, tbuf0, tbuf1, wbuf0, wbuf1,
             rbuf0, rbuf1, pay, acc,
             semg0, semg1, seml0, seml1, semj0, semj1, semi0, semi1):
    c = lax.axis_index("c")
    s = lax.axis_index("s")
    rows = zeros.shape[0]
    epw = jj.shape[0] // NS
    nchunk = epw // CH
    base = s * epw
    bufs = (
        (jvb0, ivb0, tbuf0, wbuf0, rbuf0, semg0, seml0, semj0, semi0),
        (jvb1, ivb1, tbuf1, wbuf1, rbuf1, semg1, seml1, semj1, semi1),
    )

    def task(t_t, w_t, out_t):
        plsc.subcore_barrier()
        pltpu.sync_copy(zeros, acc.at[pl.ds(s * rows, rows)])
        plsc.subcore_barrier()

        def issue(k, b):
            jvb, ivb, tbuf, wbuf, rbuf, semg, seml, semj, semi = bufs[b]
            off = base + k * CH
            pltpu.async_copy(t_t.at[jvb], tbuf, semg)
            pltpu.async_copy(w_t.at[pl.ds(off, CH)], wbuf, seml)
            pltpu.async_copy(rr.at[pl.ds(off, CH)], rbuf, seml)

        def compute(b):
            jvb, ivb, tbuf, wbuf, rbuf, semg, seml, semj, semi = bufs[b]

            @plsc.parallel_loop(0, CH, unroll=4)
            def _(e):
                rd = [rbuf[e, pl.ds(16 * d, 16)] for d in range(3)]
                for u in range(2):
                    o = 16 * u
                    s1v = tbuf[e, pl.ds(96 + o, 16)]
                    sw2 = (tbuf[e, pl.ds(128 + o, 16)]
                           * wbuf[e, pl.ds(32 + o, 16)])
                    sw3 = (tbuf[e, pl.ds(160 + o, 16)]
                           * wbuf[e, pl.ds(64 + o, 16)])
                    pay[e, pl.ds(o, 16)] = s1v * wbuf[e, pl.ds(o, 16)]
                    for d in range(3):
                        pay[e, pl.ds(32 + 32 * d + o, 16)] = (
                            tbuf[e, pl.ds(32 * d + o, 16)] * sw2
                            + rd[d] * sw3)

        def finish(k, b):
            jvb, ivb, tbuf, wbuf, rbuf, semg, seml, semj, semi = bufs[b]
            pltpu.make_async_copy(t_t.at[jvb], tbuf, semg).wait()
            pltpu.make_async_copy(w_t.at[pl.ds(0, CH)], wbuf, seml).wait()
            pltpu.make_async_copy(rr.at[pl.ds(0, CH)], rbuf, seml).wait()

            @pl.when(k + 2 < nchunk)
            def _():
                pltpu.async_copy(jj.at[pl.ds(base + (k + 2) * CH, CH)],
                                 jvb, semj)

            compute(b)
            pltpu.make_async_copy(ii.at[pl.ds(0, CH)], ivb, semi).wait()
            pltpu.sync_copy(pay, acc.at[ivb], add=True)

            @pl.when(k + 2 < nchunk)
            def _():
                pltpu.async_copy(ii.at[pl.ds(base + (k + 2) * CH, CH)],
                                 ivb, semi)
                pltpu.make_async_copy(jj.at[pl.ds(0, CH)], jvb, semj).wait()
                issue(k + 2, b)

        for b in range(2):
            jvb, ivb = bufs[b][0], bufs[b][1]
            semj, semi = bufs[b][7], bufs[b][8]
            pltpu.async_copy(jj.at[pl.ds(base + b * CH, CH)], jvb, semj)
            pltpu.async_copy(ii.at[pl.ds(base + b * CH, CH)], ivb, semi)
            pltpu.make_async_copy(jj.at[pl.ds(0, CH)], jvb, semj).wait()
            issue(b, b)

        def pair(g, carry):
            finish(2 * g, 0)
            finish(2 * g + 1, 1)
            return carry

        lax.fori_loop(0, nchunk // 2, pair, 0)
        plsc.subcore_barrier()
        pltpu.sync_copy(acc.at[pl.ds(s * rows, rows)],
                        out_t.at[pl.ds(s * rows, rows)])

    @pl.when(c == 0)
    def _():
        task(t0, w0, out0)
        task(t2, w2, out2)

    @pl.when(c == 1)
    def _():
        task(t1, w1, out1)
        task(t3, w3, out3)


def kernel(node, equivariant, rbf, envelope, r_ij, edge_index,
           Ws, bs, Wphi, bphi, Ww, bw):
    n, units = node.shape
    e = rbf.shape[0]
    f = jnp.float32

    nb = 2000
    t_tabs = pl.pallas_call(
        _mlp_body,
        grid=(n // nb,),
        in_specs=[
            pl.BlockSpec((nb, units), lambda i: (i, 0)),
            pl.BlockSpec((nb, 3, units), lambda i: (i, 0, 0)),
            pl.BlockSpec((units, units), lambda i: (0, 0)),
            pl.BlockSpec((units,), lambda i: (0,)),
            pl.BlockSpec((3 * units, units), lambda i: (0, 0)),
            pl.BlockSpec((3 * units,), lambda i: (0,)),
        ],
        out_specs=[pl.BlockSpec((nb, 256), lambda i: (i, 0))] * 4,
        out_shape=[jax.ShapeDtypeStruct((n, 256), f)] * 4,
    )(node, equivariant, Ws, bs, Wphi, bphi)

    eb = 4000
    nr = rbf.shape[1]
    w_tabs = pl.pallas_call(
        _w_body,
        grid=(e // eb,),
        in_specs=[
            pl.BlockSpec((eb, nr), lambda i: (i, 0)),
            pl.BlockSpec((eb, 1), lambda i: (i, 0)),
            pl.BlockSpec((eb, 3), lambda i: (i, 0)),
            pl.BlockSpec((3 * units, nr), lambda i: (0, 0)),
            pl.BlockSpec((3 * units,), lambda i: (0,)),
        ],
        out_specs=[pl.BlockSpec((eb, 96), lambda i: (i, 0))] * 4
        + [pl.BlockSpec((eb, 48), lambda i: (i, 0))],
        out_shape=[jax.ShapeDtypeStruct((e, 96), f)] * 4
        + [jax.ShapeDtypeStruct((e, 48), f)],
    )(rbf, envelope, r_ij, Ww, bw)
    w0, w1, w2, w3, rr = w_tabs

    ii = edge_index[:, 0]
    jj = edge_index[:, 1]
    npad = ((n + NS * 8 - 1) // (NS * 8)) * NS * 8
    zeros = jnp.zeros((npad // NS, 128), f)

    mesh = plsc.VectorSubcoreMesh(core_axis_name="c", subcore_axis_name="s")
    sc_call = pl.kernel(
        _sc_body,
        out_type=tuple(jax.ShapeDtypeStruct((npad, 128), f) for _ in range(4)),
        mesh=mesh,
        scratch_types=[
            pltpu.VMEM((CH,), jnp.int32),
            pltpu.VMEM((CH,), jnp.int32),
            pltpu.VMEM((CH,), jnp.int32),
            pltpu.VMEM((CH,), jnp.int32),
            pltpu.VMEM((CH, 256), f),
            pltpu.VMEM((CH, 256), f),
            pltpu.VMEM((CH, 96), f),
            pltpu.VMEM((CH, 96), f),
            pltpu.VMEM((CH, 48), f),
            pltpu.VMEM((CH, 48), f),
            pltpu.VMEM((CH, 128), f),
            pltpu.VMEM_SHARED((npad, 128), f),
        ] + [pltpu.SemaphoreType.DMA] * 8,
    )
    outs = sc_call(jj, ii, rr, zeros, *t_tabs, w0, w1, w2, w3)

    ds = jnp.concatenate([outs[q][:n, 0:32] for q in range(4)], axis=1)
    dv = jnp.stack(
        [jnp.concatenate([outs[q][:n, 32 + 32 * d:64 + 32 * d]
                          for q in range(4)], axis=1) for d in range(3)],
        axis=1)
    return (ds, dv)


# speedup vs baseline: 18.2437x; 1.0773x over previous
import jax
import jax.numpy as jnp
from jax import lax
from jax.experimental import pallas as pl
from jax.experimental.pallas import tpu as pltpu
from jax.experimental.pallas import tpu_sc as plsc

NC = 2
NS = 16
CH = 40


def _mlp_body(node_ref, eqf_ref, Ws_ref, bs_ref, Wphi_ref, bphi_ref, *outs):
    x = node_ref[...]
    h = lax.dot_general(x, Ws_ref[...], (((1,), (1,)), ((), ())),
                        preferred_element_type=jnp.float32)
    h = h + bs_ref[...][None, :]
    h = h * jax.nn.sigmoid(h)
    s = lax.dot_general(h, Wphi_ref[...], (((1,), (1,)), ((), ())),
                        preferred_element_type=jnp.float32)
    s = s + bphi_ref[...][None, :]
    eqf = eqf_ref[...]
    pad = jnp.zeros((x.shape[0], 64), jnp.float32)
    for q in range(4):
        o = 32 * q
        outs[q][...] = jnp.concatenate(
            [eqf[:, 0, o:o + 32], eqf[:, 1, o:o + 32], eqf[:, 2, o:o + 32],
             s[:, o:o + 32], s[:, 128 + o:128 + o + 32],
             s[:, 256 + o:256 + o + 32], pad], axis=1)


def _w_body(rbf_ref, env_ref, rij_ref, m0, m1, m2, m3, b0, b1, b2, b3, bb,
            w0, w1, w2, w3, r_out):
    rbf = rbf_ref[...]
    env = env_ref[...]
    ms = (m0, m1, m2, m3)
    bs_ = (b0, b1, b2, b3)
    ws = (w0, w1, w2, w3)
    for q in range(4):
        wq = lax.dot_general(rbf, ms[q][...], (((1,), (0,)), ((), ())),
                             preferred_element_type=jnp.float32)
        ws[q][...] = (wq + bs_[q][...][None, :]) * env
    r_out[...] = lax.dot_general(rij_ref[...], bb[...],
                                 (((1,), (0,)), ((), ())),
                                 precision=lax.Precision.HIGHEST,
                                 preferred_element_type=jnp.float32)


def _sc_body(jj, ii, rr, zeros, t0, t1, t2, t3, w0, w1, w2, w3,
             out0, out1, out2, out3,
             jvb0, jvb1, ivb0, ivb1, tbuf0, tbuf1, wbuf0, wbuf1,
             rbuf0, rbuf1, pay, acc,
             semg0, semg1, seml0, seml1, semj0, semj1, semi0, semi1):
    c = lax.axis_index("c")
    s = lax.axis_index("s")
    rows = zeros.shape[0]
    epw = jj.shape[0] // NS
    nchunk = epw // CH
    base = s * epw
    bufs = (
        (jvb0, ivb0, tbuf0, wbuf0, rbuf0, semg0, seml0, semj0, semi0),
        (jvb1, ivb1, tbuf1, wbuf1, rbuf1, semg1, seml1, semj1, semi1),
    )

    def task(t_t, w_t, out_t):
        plsc.subcore_barrier()
        pltpu.sync_copy(zeros, acc.at[pl.ds(s * rows, rows)])
        plsc.subcore_barrier()

        def issue(k, b):
            jvb, ivb, tbuf, wbuf, rbuf, semg, seml, semj, semi = bufs[b]
            off = base + k * CH
            pltpu.async_copy(t_t.at[jvb], tbuf, semg)
            pltpu.async_copy(w_t.at[pl.ds(off, CH)], wbuf, seml)
            pltpu.async_copy(rr.at[pl.ds(off, CH)], rbuf, seml)

        def compute(b):
            jvb, ivb, tbuf, wbuf, rbuf, semg, seml, semj, semi = bufs[b]

            @plsc.parallel_loop(0, CH, unroll=4)
            def _(e):
                rd = [rbuf[e, pl.ds(16 * d, 16)] for d in range(3)]
                for u in range(2):
                    o = 16 * u
                    s1v = tbuf[e, pl.ds(96 + o, 16)]
                    sw2 = (tbuf[e, pl.ds(128 + o, 16)]
                           * wbuf[e, pl.ds(32 + o, 16)])
                    sw3 = (tbuf[e, pl.ds(160 + o, 16)]
                           * wbuf[e, pl.ds(64 + o, 16)])
                    pay[e, pl.ds(o, 16)] = s1v * wbuf[e, pl.ds(o, 16)]
                    for d in range(3):
                        pay[e, pl.ds(32 + 32 * d + o, 16)] = (
                            tbuf[e, pl.ds(32 * d + o, 16)] * sw2
                            + rd[d] * sw3)

        def finish(k, b):
            jvb, ivb, tbuf, wbuf, rbuf, semg, seml, semj, semi = bufs[b]
            pltpu.make_async_copy(t_t.at[jvb], tbuf, semg).wait()
            pltpu.make_async_copy(w_t.at[pl.ds(0, CH)], wbuf, seml).wait()
            pltpu.make_async_copy(rr.at[pl.ds(0, CH)], rbuf, seml).wait()

            @pl.when(k + 2 < nchunk)
            def _():
                pltpu.async_copy(jj.at[pl.ds(base + (k + 2) * CH, CH)],
                                 jvb, semj)

            compute(b)
            pltpu.make_async_copy(ii.at[pl.ds(0, CH)], ivb, semi).wait()
            pltpu.sync_copy(pay, acc.at[ivb], add=True)

            @pl.when(k + 2 < nchunk)
            def _():
                pltpu.async_copy(ii.at[pl.ds(base + (k + 2) * CH, CH)],
                                 ivb, semi)
                pltpu.make_async_copy(jj.at[pl.ds(0, CH)], jvb, semj).wait()
                issue(k + 2, b)

        for b in range(2):
            jvb, ivb = bufs[b][0], bufs[b][1]
            semj, semi = bufs[b][7], bufs[b][8]
            pltpu.async_copy(jj.at[pl.ds(base + b * CH, CH)], jvb, semj)
            pltpu.async_copy(ii.at[pl.ds(base + b * CH, CH)], ivb, semi)
            pltpu.make_async_copy(jj.at[pl.ds(0, CH)], jvb, semj).wait()
            issue(b, b)

        def pair(g, carry):
            finish(2 * g, 0)
            finish(2 * g + 1, 1)
            return carry

        lax.fori_loop(0, nchunk // 2, pair, 0)
        plsc.subcore_barrier()
        pltpu.sync_copy(acc.at[pl.ds(s * rows, rows)],
                        out_t.at[pl.ds(s * rows, rows)])

    @pl.when(c == 0)
    def _():
        task(t0, w0, out0)
        task(t2, w2, out2)

    @pl.when(c == 1)
    def _():
        task(t1, w1, out1)
        task(t3, w3, out3)


def kernel(node, equivariant, rbf, envelope, r_ij, edge_index,
           Ws, bs, Wphi, bphi, Ww, bw):
    n, units = node.shape
    e = rbf.shape[0]
    f = jnp.float32

    nb = 2000
    t_tabs = pl.pallas_call(
        _mlp_body,
        grid=(n // nb,),
        in_specs=[
            pl.BlockSpec((nb, units), lambda i: (i, 0)),
            pl.BlockSpec((nb, 3, units), lambda i: (i, 0, 0)),
            pl.BlockSpec((units, units), lambda i: (0, 0)),
            pl.BlockSpec((units,), lambda i: (0,)),
            pl.BlockSpec((3 * units, units), lambda i: (0, 0)),
            pl.BlockSpec((3 * units,), lambda i: (0,)),
        ],
        out_specs=[pl.BlockSpec((nb, 256), lambda i: (i, 0))] * 4,
        out_shape=[jax.ShapeDtypeStruct((n, 256), f)] * 4,
    )(node, equivariant, Ws, bs, Wphi, bphi)

    eb = 4000
    nr = rbf.shape[1]
    wwt = Ww.T
    mws = [jnp.concatenate([wwt[:, o:o + 32], wwt[:, 128 + o:128 + o + 32],
                            wwt[:, 256 + o:256 + o + 32]], axis=1)
           for o in (0, 32, 64, 96)]
    bws = [jnp.concatenate([bw[o:o + 32], bw[128 + o:128 + o + 32],
                            bw[256 + o:256 + o + 32]])
           for o in (0, 32, 64, 96)]
    bmat = jnp.repeat(jnp.eye(3, dtype=f), 16, axis=1)
    w_tabs = pl.pallas_call(
        _w_body,
        grid=(e // eb,),
        in_specs=[
            pl.BlockSpec((eb, nr), lambda i: (i, 0)),
            pl.BlockSpec((eb, 1), lambda i: (i, 0)),
            pl.BlockSpec((eb, 3), lambda i: (i, 0)),
        ] + [pl.BlockSpec((nr, 96), lambda i: (0, 0))] * 4
        + [pl.BlockSpec((96,), lambda i: (0,))] * 4
        + [pl.BlockSpec((3, 48), lambda i: (0, 0))],
        out_specs=[pl.BlockSpec((eb, 96), lambda i: (i, 0))] * 4
        + [pl.BlockSpec((eb, 48), lambda i: (i, 0))],
        out_shape=[jax.ShapeDtypeStruct((e, 96), f)] * 4
        + [jax.ShapeDtypeStruct((e, 48), f)],
    )(rbf, envelope, r_ij, *mws, *bws, bmat)
    w0, w1, w2, w3, rr = w_tabs

    ii = edge_index[:, 0]
    jj = edge_index[:, 1]
    npad = ((n + NS * 8 - 1) // (NS * 8)) * NS * 8
    zeros = jnp.zeros((npad // NS, 128), f)

    mesh = plsc.VectorSubcoreMesh(core_axis_name="c", subcore_axis_name="s")
    sc_call = pl.kernel(
        _sc_body,
        out_type=tuple(jax.ShapeDtypeStruct((npad, 128), f) for _ in range(4)),
        mesh=mesh,
        scratch_types=[
            pltpu.VMEM((CH,), jnp.int32),
            pltpu.VMEM((CH,), jnp.int32),
            pltpu.VMEM((CH,), jnp.int32),
            pltpu.VMEM((CH,), jnp.int32),
            pltpu.VMEM((CH, 256), f),
            pltpu.VMEM((CH, 256), f),
            pltpu.VMEM((CH, 96), f),
            pltpu.VMEM((CH, 96), f),
            pltpu.VMEM((CH, 48), f),
            pltpu.VMEM((CH, 48), f),
            pltpu.VMEM((CH, 128), f),
            pltpu.VMEM_SHARED((npad, 128), f),
        ] + [pltpu.SemaphoreType.DMA] * 8,
    )
    outs = sc_call(jj, ii, rr, zeros, *t_tabs, w0, w1, w2, w3)

    ds = jnp.concatenate([outs[q][:n, 0:32] for q in range(4)], axis=1)
    dv = jnp.stack(
        [jnp.concatenate([outs[q][:n, 32 + 32 * d:64 + 32 * d]
                          for q in range(4)], axis=1) for d in range(3)],
        axis=1)
    return (ds, dv)
